# speedup vs baseline: 3.0862x; 3.0862x over previous
import functools

import jax
import jax.numpy as jnp
from jax.experimental import pallas as pl
from jax.experimental.pallas import tpu as pltpu

_TEMP_INV = 10.0
_K = 8
_NEG = -3.0e38


def _topk_merge(vals, pays, k):
    R, W = vals.shape
    iota = jax.lax.broadcasted_iota(jnp.int32, (R, W), 1)
    out_v, out_p = [], []
    for _ in range(k):
        mx = jnp.max(vals, axis=1, keepdims=True)
        ismx = vals == mx
        first = jnp.min(jnp.where(ismx, iota, W), axis=1, keepdims=True)
        sel = iota == first
        out_v.append(mx)
        out_p.append(jnp.sum(jnp.where(sel, pays, 0.0), axis=1, keepdims=True))
        vals = jnp.where(sel, _NEG, vals)
    return jnp.concatenate(out_v, axis=1), jnp.concatenate(out_p, axis=1)


def _body(zr_ref, zc_ref, tr_ref, tc_ref, out_ref,
          m_ref, s_ref, tv_ref, tp_ref, *, rtile, ctile, batch):
    r = pl.program_id(0)
    c = pl.program_id(1)
    nc = pl.num_programs(1)

    @pl.when(c == 0)
    def _init_row_state():
        m_ref[...] = jnp.full_like(m_ref, _NEG)
        s_ref[...] = jnp.zeros_like(s_ref)
        tv_ref[...] = jnp.full_like(tv_ref, _NEG)
        tp_ref[...] = jnp.zeros_like(tp_ref)

    @pl.when((r == 0) & (c == 0))
    def _init_out():
        out_ref[...] = jnp.zeros_like(out_ref)

    def _normalize(x):
        ss = jnp.sum(x * x, axis=1, keepdims=True)
        return x / jnp.maximum(jnp.sqrt(ss), 1e-12)

    zr = _normalize(zr_ref[...])
    zc = _normalize(zc_ref[...])
    tr = _normalize(tr_ref[...])
    tc = _normalize(tc_ref[...])

    dims = (((1,), (1,)), ((), ()))
    logits = jax.lax.dot_general(
        zr, zc, dims, preferred_element_type=jnp.float32) * _TEMP_INV
    topo = jax.lax.dot_general(
        tr, tc, dims, preferred_element_type=jnp.float32)

    row_g = r * rtile + jax.lax.broadcasted_iota(jnp.int32, (rtile, ctile), 0)
    col_g = c * ctile + jax.lax.broadcasted_iota(jnp.int32, (rtile, ctile), 1)
    diag = row_g == col_g
    logits = jnp.where(diag, -1e9, logits)
    topo = jnp.where(diag, _NEG, topo)

    m_old = m_ref[...]
    m_new = jnp.maximum(m_old, jnp.max(logits, axis=1, keepdims=True))
    s_ref[...] = (s_ref[...] * jnp.exp(m_old - m_new)
                  + jnp.sum(jnp.exp(logits - m_new), axis=1, keepdims=True))
    m_ref[...] = m_new

    tile_v, tile_p = _topk_merge(topo, logits, _K)
    merged_v = jnp.concatenate([tv_ref[...], tile_v], axis=1)
    merged_p = jnp.concatenate([tp_ref[...], tile_p], axis=1)
    mv, mp = _topk_merge(merged_v, merged_p, _K)
    tv_ref[...] = mv
    tp_ref[...] = mp

    @pl.when(c == nc - 1)
    def _finish_rows():
        lse = m_ref[...] + jnp.log(s_ref[...])
        pos = jnp.sum(tp_ref[...], axis=1, keepdims=True) / _K
        part = jnp.sum(lse - pos, axis=0, keepdims=True)
        out_ref[...] += part / batch


def kernel(projections, topo_vectors):
    B, D = projections.shape
    Dt = topo_vectors.shape[1]
    rtile, ctile = 256, 512
    nr, nc = B // rtile, B // ctile

    out = pl.pallas_call(
        functools.partial(_body, rtile=rtile, ctile=ctile, batch=float(B)),
        grid=(nr, nc),
        in_specs=[
            pl.BlockSpec((rtile, D), lambda r, c: (r, 0)),
            pl.BlockSpec((ctile, D), lambda r, c: (c, 0)),
            pl.BlockSpec((rtile, Dt), lambda r, c: (r, 0)),
            pl.BlockSpec((ctile, Dt), lambda r, c: (c, 0)),
        ],
        out_specs=pl.BlockSpec((1, 1), lambda r, c: (0, 0)),
        out_shape=jax.ShapeDtypeStruct((1, 1), jnp.float32),
        scratch_shapes=[
            pltpu.VMEM((rtile, 1), jnp.float32),
            pltpu.VMEM((rtile, 1), jnp.float32),
            pltpu.VMEM((rtile, _K), jnp.float32),
            pltpu.VMEM((rtile, _K), jnp.float32),
        ],
        compiler_params=pltpu.CompilerParams(
            dimension_semantics=("arbitrary", "arbitrary")),
    )(projections, projections, topo_vectors, topo_vectors)
    return out[0, 0]


# speedup vs baseline: 5.3487x; 1.7331x over previous
import functools

import jax
import jax.numpy as jnp
from jax.experimental import pallas as pl
from jax.experimental.pallas import tpu as pltpu

_TEMP_INV = 10.0
_K = 8
_NEG = -3.0e38


def _topk_merge(vals, pays, k):
    out_v, out_p = [], []
    for _ in range(k):
        mx = jnp.max(vals, axis=1, keepdims=True)
        sel = vals == mx
        out_v.append(mx)
        out_p.append(jnp.sum(jnp.where(sel, pays, 0.0), axis=1, keepdims=True))
        vals = jnp.where(sel, _NEG, vals)
    return jnp.concatenate(out_v, axis=1), jnp.concatenate(out_p, axis=1)


def _body(zr_ref, zc_ref, tr_ref, tc_ref, out_ref,
          m_ref, s_ref, tv_ref, tp_ref, *, rtile, ctile, batch):
    r = pl.program_id(0)
    c = pl.program_id(1)
    nc = pl.num_programs(1)

    @pl.when(c == 0)
    def _init_row_state():
        m_ref[...] = jnp.full_like(m_ref, _NEG)
        s_ref[...] = jnp.zeros_like(s_ref)
        tv_ref[...] = jnp.full_like(tv_ref, _NEG)
        tp_ref[...] = jnp.zeros_like(tp_ref)

    @pl.when((r == 0) & (c == 0))
    def _init_out():
        out_ref[...] = jnp.zeros_like(out_ref)

    def _normalize(x):
        ss = jnp.sum(x * x, axis=1, keepdims=True)
        return x / jnp.maximum(jnp.sqrt(ss), 1e-12)

    zr = _normalize(zr_ref[...])
    zc = _normalize(zc_ref[...])
    tr = _normalize(tr_ref[...])
    tc = _normalize(tc_ref[...])

    dims = (((1,), (1,)), ((), ()))
    logits = jax.lax.dot_general(
        zr, zc, dims, preferred_element_type=jnp.float32) * _TEMP_INV
    topo = jax.lax.dot_general(
        tr, tc, dims, preferred_element_type=jnp.float32)

    row_g = r * rtile + jax.lax.broadcasted_iota(jnp.int32, (rtile, ctile), 0)
    col_g = c * ctile + jax.lax.broadcasted_iota(jnp.int32, (rtile, ctile), 1)
    diag = row_g == col_g
    logits = jnp.where(diag, -1e9, logits)
    topo = jnp.where(diag, _NEG, topo)

    m_old = m_ref[...]
    m_new = jnp.maximum(m_old, jnp.max(logits, axis=1, keepdims=True))
    s_ref[...] = (s_ref[...] * jnp.exp(m_old - m_new)
                  + jnp.sum(jnp.exp(logits - m_new), axis=1, keepdims=True))
    m_ref[...] = m_new

    tile_v, tile_p = _topk_merge(topo, logits, _K)
    merged_v = jnp.concatenate([tv_ref[...], tile_v], axis=1)
    merged_p = jnp.concatenate([tp_ref[...], tile_p], axis=1)
    mv, mp = _topk_merge(merged_v, merged_p, _K)
    tv_ref[...] = mv
    tp_ref[...] = mp

    @pl.when(c == nc - 1)
    def _finish_rows():
        lse = m_ref[...] + jnp.log(s_ref[...])
        pos = jnp.sum(tp_ref[...], axis=1, keepdims=True) / _K
        part = jnp.sum(lse - pos, axis=0, keepdims=True)
        out_ref[...] += part / batch


def kernel(projections, topo_vectors):
    B, D = projections.shape
    Dt = topo_vectors.shape[1]
    rtile, ctile = 256, 512
    nr, nc = B // rtile, B // ctile

    out = pl.pallas_call(
        functools.partial(_body, rtile=rtile, ctile=ctile, batch=float(B)),
        grid=(nr, nc),
        in_specs=[
            pl.BlockSpec((rtile, D), lambda r, c: (r, 0)),
            pl.BlockSpec((ctile, D), lambda r, c: (c, 0)),
            pl.BlockSpec((rtile, Dt), lambda r, c: (r, 0)),
            pl.BlockSpec((ctile, Dt), lambda r, c: (c, 0)),
        ],
        out_specs=pl.BlockSpec((1, 1), lambda r, c: (0, 0)),
        out_shape=jax.ShapeDtypeStruct((1, 1), jnp.float32),
        scratch_shapes=[
            pltpu.VMEM((rtile, 1), jnp.float32),
            pltpu.VMEM((rtile, 1), jnp.float32),
            pltpu.VMEM((rtile, _K), jnp.float32),
            pltpu.VMEM((rtile, _K), jnp.float32),
        ],
        compiler_params=pltpu.CompilerParams(
            dimension_semantics=("arbitrary", "arbitrary")),
    )(projections, projections, topo_vectors, topo_vectors)
    return out[0, 0]


# speedup vs baseline: 6.4727x; 1.2102x over previous
import functools

import jax
import jax.numpy as jnp
from jax.experimental import pallas as pl
from jax.experimental.pallas import tpu as pltpu

_TEMP_INV = 10.0
_K = 8
_NEG = -3.0e38


def _normalize_body(z_ref, t_ref, zo_ref, to_ref):
    z = z_ref[...]
    t = t_ref[...]
    zn = jnp.maximum(jnp.sqrt(jnp.sum(z * z, axis=1, keepdims=True)), 1e-12)
    tn = jnp.maximum(jnp.sqrt(jnp.sum(t * t, axis=1, keepdims=True)), 1e-12)
    zo_ref[...] = z / zn
    to_ref[...] = t / tn


def _topk_merge(vals, pays, k):
    out_v, out_p = [], []
    for _ in range(k):
        mx = jnp.max(vals, axis=1, keepdims=True)
        sel = vals == mx
        out_v.append(mx)
        out_p.append(jnp.sum(jnp.where(sel, pays, 0.0), axis=1, keepdims=True))
        vals = jnp.where(sel, _NEG, vals)
    return jnp.concatenate(out_v, axis=1), jnp.concatenate(out_p, axis=1)


def _body(zr_ref, zc_ref, tr_ref, tc_ref, out_ref,
          s_ref, tv_ref, tp_ref, *, rtile, ctile, batch):
    r = pl.program_id(0)
    c = pl.program_id(1)
    nc = pl.num_programs(1)

    @pl.when(c == 0)
    def _init_row_state():
        s_ref[...] = jnp.zeros_like(s_ref)
        tv_ref[...] = jnp.full_like(tv_ref, _NEG)
        tp_ref[...] = jnp.zeros_like(tp_ref)

    @pl.when((r == 0) & (c == 0))
    def _init_out():
        out_ref[...] = jnp.zeros_like(out_ref)

    dims = (((1,), (1,)), ((), ()))
    logits = jax.lax.dot_general(
        zr_ref[...], zc_ref[...], dims,
        preferred_element_type=jnp.float32) * _TEMP_INV
    topo = jax.lax.dot_general(
        tr_ref[...], tc_ref[...], dims, preferred_element_type=jnp.float32)

    row_g = r * rtile + jax.lax.broadcasted_iota(jnp.int32, (rtile, ctile), 0)
    col_g = c * ctile + jax.lax.broadcasted_iota(jnp.int32, (rtile, ctile), 1)
    diag = row_g == col_g
    logits = jnp.where(diag, -1e9, logits)
    topo = jnp.where(diag, _NEG, topo)

    s_ref[...] += jnp.sum(jnp.exp(logits - _TEMP_INV), axis=1, keepdims=True)

    tile_v, tile_p = _topk_merge(topo, logits, _K)
    merged_v = jnp.concatenate([tv_ref[...], tile_v], axis=1)
    merged_p = jnp.concatenate([tp_ref[...], tile_p], axis=1)
    mv, mp = _topk_merge(merged_v, merged_p, _K)
    tv_ref[...] = mv
    tp_ref[...] = mp

    @pl.when(c == nc - 1)
    def _finish_rows():
        lse = _TEMP_INV + jnp.log(s_ref[...])
        pos = jnp.sum(tp_ref[...], axis=1, keepdims=True) / _K
        part = jnp.sum(lse - pos, axis=0, keepdims=True)
        out_ref[...] += part / batch


def kernel(projections, topo_vectors):
    B, D = projections.shape
    Dt = topo_vectors.shape[1]

    z, topo_z = pl.pallas_call(
        _normalize_body,
        out_shape=[
            jax.ShapeDtypeStruct((B, D), jnp.float32),
            jax.ShapeDtypeStruct((B, Dt), jnp.float32),
        ],
    )(projections, topo_vectors)

    rtile, ctile = 256, 512
    nr, nc = B // rtile, B // ctile

    out = pl.pallas_call(
        functools.partial(_body, rtile=rtile, ctile=ctile, batch=float(B)),
        grid=(nr, nc),
        in_specs=[
            pl.BlockSpec((rtile, D), lambda r, c: (r, 0)),
            pl.BlockSpec((ctile, D), lambda r, c: (c, 0)),
            pl.BlockSpec((rtile, Dt), lambda r, c: (r, 0)),
            pl.BlockSpec((ctile, Dt), lambda r, c: (c, 0)),
        ],
        out_specs=pl.BlockSpec((1, 1), lambda r, c: (0, 0)),
        out_shape=jax.ShapeDtypeStruct((1, 1), jnp.float32),
        scratch_shapes=[
            pltpu.VMEM((rtile, 1), jnp.float32),
            pltpu.VMEM((rtile, _K), jnp.float32),
            pltpu.VMEM((rtile, _K), jnp.float32),
        ],
        compiler_params=pltpu.CompilerParams(
            dimension_semantics=("arbitrary", "arbitrary")),
    )(z, z, topo_z, topo_z)
    return out[0, 0]


# speedup vs baseline: 8.9505x; 1.3828x over previous
import functools

import jax
import jax.numpy as jnp
from jax.experimental import pallas as pl
from jax.experimental.pallas import tpu as pltpu

_TEMP_INV = 10.0
_K = 8
_NEG = -3.0e38


def _normalize_body(z_ref, t_ref, zo_ref, to_ref):
    z = z_ref[...]
    t = t_ref[...]
    zn = jnp.maximum(jnp.sqrt(jnp.sum(z * z, axis=1, keepdims=True)), 1e-12)
    tn = jnp.maximum(jnp.sqrt(jnp.sum(t * t, axis=1, keepdims=True)), 1e-12)
    zo_ref[...] = z / zn
    to_ref[...] = t / tn


def _body(zr_ref, zc_ref, tr_ref, tc_ref, out_ref,
          sacc_ref, pacc_ref, tv_ref, *, rtile, ctile, batch):
    r = pl.program_id(0)
    p = pl.program_id(1)
    c = pl.program_id(2)
    nc = pl.num_programs(2)

    dims = (((1,), (1,)), ((), ()))
    row_g = r * rtile + jax.lax.broadcasted_iota(jnp.int32, (rtile, ctile), 0)
    col_g = c * ctile + jax.lax.broadcasted_iota(jnp.int32, (rtile, ctile), 1)
    diag = row_g == col_g

    @pl.when(p == 0)
    def _phase_rank():
        @pl.when(c == 0)
        def _init_tv():
            tv_ref[...] = jnp.full_like(tv_ref, _NEG)

        topo = jax.lax.dot_general(
            tr_ref[...], tc_ref[...], dims, preferred_element_type=jnp.float32)
        topo = jnp.where(diag, _NEG, topo)

        vals = jnp.concatenate([topo, tv_ref[...]], axis=1)
        prev = jnp.full((rtile, 1), jnp.inf, dtype=jnp.float32)
        out_v = []
        for _ in range(_K):
            masked = jnp.where(vals < prev, vals, _NEG)
            prev = jnp.max(masked, axis=1, keepdims=True)
            out_v.append(prev)
        tv_ref[...] = jnp.concatenate(out_v, axis=1)

    @pl.when(p == 1)
    def _phase_accumulate():
        @pl.when(c == 0)
        def _init_acc():
            sacc_ref[...] = jnp.zeros_like(sacc_ref)
            pacc_ref[...] = jnp.zeros_like(pacc_ref)

        @pl.when((r == 0) & (c == 0))
        def _init_out():
            out_ref[...] = jnp.zeros_like(out_ref)

        logits = jax.lax.dot_general(
            zr_ref[...], zc_ref[...], dims,
            preferred_element_type=jnp.float32) * _TEMP_INV
        topo = jax.lax.dot_general(
            tr_ref[...], tc_ref[...], dims, preferred_element_type=jnp.float32)
        logits = jnp.where(diag, -1e9, logits)
        topo = jnp.where(diag, _NEG, topo)

        sacc_ref[...] += jnp.exp(logits - _TEMP_INV)
        t8 = tv_ref[:, _K - 1:_K]
        pacc_ref[...] += jnp.where(topo >= t8, logits, 0.0)

        @pl.when(c == nc - 1)
        def _finish_rows():
            srow = jnp.sum(sacc_ref[...], axis=1, keepdims=True)
            prow = jnp.sum(pacc_ref[...], axis=1, keepdims=True)
            lse = _TEMP_INV + jnp.log(srow)
            part = jnp.sum(lse - prow / _K, axis=0, keepdims=True)
            out_ref[...] += part / batch


def kernel(projections, topo_vectors):
    B, D = projections.shape
    Dt = topo_vectors.shape[1]

    z, topo_z = pl.pallas_call(
        _normalize_body,
        out_shape=[
            jax.ShapeDtypeStruct((B, D), jnp.float32),
            jax.ShapeDtypeStruct((B, Dt), jnp.float32),
        ],
    )(projections, topo_vectors)

    rtile, ctile = 256, 512
    nr, nc = B // rtile, B // ctile

    out = pl.pallas_call(
        functools.partial(_body, rtile=rtile, ctile=ctile, batch=float(B)),
        grid=(nr, 2, nc),
        in_specs=[
            pl.BlockSpec((rtile, D), lambda r, p, c: (r, 0)),
            pl.BlockSpec((ctile, D), lambda r, p, c: (c, 0)),
            pl.BlockSpec((rtile, Dt), lambda r, p, c: (r, 0)),
            pl.BlockSpec((ctile, Dt), lambda r, p, c: (c, 0)),
        ],
        out_specs=pl.BlockSpec((1, 1), lambda r, p, c: (0, 0)),
        out_shape=jax.ShapeDtypeStruct((1, 1), jnp.float32),
        scratch_shapes=[
            pltpu.VMEM((rtile, ctile), jnp.float32),
            pltpu.VMEM((rtile, ctile), jnp.float32),
            pltpu.VMEM((rtile, _K), jnp.float32),
        ],
        compiler_params=pltpu.CompilerParams(
            dimension_semantics=("arbitrary", "arbitrary", "arbitrary")),
    )(z, z, topo_z, topo_z)
    return out[0, 0]


# speedup vs baseline: 21.0633x; 2.3533x over previous
import functools

import jax
import jax.numpy as jnp
from jax.experimental import pallas as pl
from jax.experimental.pallas import tpu as pltpu

_TEMP_INV = 10.0
_K = 8
_NEG = -3.0e38


def _normalize_body(z_ref, t_ref, zo_ref, to_ref):
    z = z_ref[...]
    t = t_ref[...]
    zn = jnp.maximum(jnp.sqrt(jnp.sum(z * z, axis=1, keepdims=True)), 1e-12)
    tn = jnp.maximum(jnp.sqrt(jnp.sum(t * t, axis=1, keepdims=True)), 1e-12)
    zo_ref[...] = z / zn
    to_ref[...] = t / tn


def _body(zr_ref, zall_ref, tr_ref, tall_ref, out_ref, *, rtile, batch):
    r = pl.program_id(0)
    nr = pl.num_programs(0)
    B = int(batch)

    dims = (((1,), (1,)), ((), ()))
    row_g = r * rtile + jax.lax.broadcasted_iota(jnp.int32, (rtile, B), 0)
    col_g = jax.lax.broadcasted_iota(jnp.int32, (rtile, B), 1)
    diag = row_g == col_g

    @pl.when(r == 0)
    def _init_out():
        out_ref[...] = jnp.zeros_like(out_ref)

    topo = jax.lax.dot_general(
        tr_ref[...], tall_ref[...], dims, preferred_element_type=jnp.float32)
    topo = jnp.where(diag, _NEG, topo)

    prev = jnp.max(topo, axis=1, keepdims=True)
    for _ in range(_K - 1):
        masked = jnp.where(topo < prev, topo, _NEG)
        prev = jnp.max(masked, axis=1, keepdims=True)
    t8 = prev

    logits = jax.lax.dot_general(
        zr_ref[...], zall_ref[...], dims,
        preferred_element_type=jnp.float32) * _TEMP_INV
    logits = jnp.where(diag, -1e9, logits)

    srow = jnp.sum(jnp.exp(logits - _TEMP_INV), axis=1, keepdims=True)
    prow = jnp.sum(jnp.where(topo >= t8, logits, 0.0), axis=1, keepdims=True)
    lse = _TEMP_INV + jnp.log(srow)
    part = jnp.sum(lse - prow / _K, axis=0, keepdims=True)
    out_ref[...] += part / batch


def kernel(projections, topo_vectors):
    B, D = projections.shape
    Dt = topo_vectors.shape[1]

    z, topo_z = pl.pallas_call(
        _normalize_body,
        out_shape=[
            jax.ShapeDtypeStruct((B, D), jnp.float32),
            jax.ShapeDtypeStruct((B, Dt), jnp.float32),
        ],
    )(projections, topo_vectors)

    rtile = 256
    nr = B // rtile

    out = pl.pallas_call(
        functools.partial(_body, rtile=rtile, batch=float(B)),
        grid=(nr,),
        in_specs=[
            pl.BlockSpec((rtile, D), lambda r: (r, 0)),
            pl.BlockSpec((B, D), lambda r: (0, 0)),
            pl.BlockSpec((rtile, Dt), lambda r: (r, 0)),
            pl.BlockSpec((B, Dt), lambda r: (0, 0)),
        ],
        out_specs=pl.BlockSpec((1, 1), lambda r: (0, 0)),
        out_shape=jax.ShapeDtypeStruct((1, 1), jnp.float32),
        compiler_params=pltpu.CompilerParams(
            dimension_semantics=("arbitrary",)),
    )(z, z, topo_z, topo_z)
    return out[0, 0]


# speedup vs baseline: 22.1749x; 1.0528x over previous
import functools

import jax
import jax.numpy as jnp
from jax.experimental import pallas as pl
from jax.experimental.pallas import tpu as pltpu

_TEMP_INV = 10.0
_K = 8
_NEG = -3.0e38


def _normalize_body(z_ref, t_ref, zr_ref, zc_ref, to_ref):
    z = z_ref[...]
    t = t_ref[...]
    zn = jnp.maximum(jnp.sqrt(jnp.sum(z * z, axis=1, keepdims=True)), 1e-12)
    tn = jnp.maximum(jnp.sqrt(jnp.sum(t * t, axis=1, keepdims=True)), 1e-12)
    zr_ref[...] = z / zn
    zc_ref[...] = (z / zn) * _TEMP_INV
    to_ref[...] = t / tn


def _body(zr_ref, zc_ref, tr_ref, tc_ref, out_ref, *, rtile, batch):
    r = pl.program_id(0)

    @pl.when(r == 0)
    def _init_out():
        out_ref[...] = jnp.zeros_like(out_ref)

    dims = (((1,), (1,)), ((), ()))
    topo = jax.lax.dot_general(
        tr_ref[...], tc_ref[...], dims, preferred_element_type=jnp.float32)

    prev = jnp.max(topo, axis=1, keepdims=True)
    for _ in range(_K):
        masked = jnp.where(topo < prev, topo, _NEG)
        prev = jnp.max(masked, axis=1, keepdims=True)
    t8 = prev

    zr = zr_ref[...]
    logits = jax.lax.dot_general(
        zr, zc_ref[...], dims, preferred_element_type=jnp.float32)

    lii = _TEMP_INV * jnp.sum(zr * zr, axis=1, keepdims=True)

    srow = jnp.sum(jnp.exp(logits), axis=1, keepdims=True) - jnp.exp(lii)
    prow = (jnp.sum(jnp.where(topo >= t8, logits, 0.0), axis=1, keepdims=True)
            - lii)
    lse = jnp.log(srow)
    part = jnp.sum(lse - prow / _K, axis=0, keepdims=True)
    out_ref[...] += part / batch


def kernel(projections, topo_vectors):
    B, D = projections.shape
    Dt = topo_vectors.shape[1]

    zrow, zcol, topo_z = pl.pallas_call(
        _normalize_body,
        out_shape=[
            jax.ShapeDtypeStruct((B, D), jnp.float32),
            jax.ShapeDtypeStruct((B, D), jnp.float32),
            jax.ShapeDtypeStruct((B, Dt), jnp.float32),
        ],
    )(projections, topo_vectors)

    rtile = 256
    nr = B // rtile

    out = pl.pallas_call(
        functools.partial(_body, rtile=rtile, batch=float(B)),
        grid=(nr,),
        in_specs=[
            pl.BlockSpec((rtile, D), lambda r: (r, 0)),
            pl.BlockSpec((B, D), lambda r: (0, 0)),
            pl.BlockSpec((rtile, Dt), lambda r: (r, 0)),
            pl.BlockSpec((B, Dt), lambda r: (0, 0)),
        ],
        out_specs=pl.BlockSpec((1, 1), lambda r: (0, 0)),
        out_shape=jax.ShapeDtypeStruct((1, 1), jnp.float32),
        compiler_params=pltpu.CompilerParams(
            dimension_semantics=("arbitrary",)),
    )(zrow, zcol, topo_z, topo_z)
    return out[0, 0]


# speedup vs baseline: 28.6057x; 1.2900x over previous
import functools

import jax
import jax.numpy as jnp
from jax.experimental import pallas as pl
from jax.experimental.pallas import tpu as pltpu

_TEMP_INV = 10.0
_K = 8
_NEG = -3.0e38


def _normalize_body(z_ref, t_ref, zr_ref, zc_ref, to_ref):
    z = z_ref[...]
    t = t_ref[...]
    zn = jnp.maximum(jnp.sqrt(jnp.sum(z * z, axis=1, keepdims=True)), 1e-12)
    tn = jnp.maximum(jnp.sqrt(jnp.sum(t * t, axis=1, keepdims=True)), 1e-12)
    zr_ref[...] = z / zn
    zc_ref[...] = (z / zn) * _TEMP_INV
    to_ref[...] = (t / tn).astype(jnp.bfloat16)


def _body(zr_ref, zc_ref, tr_ref, tc_ref, out_ref, *, rtile, batch):
    r = pl.program_id(0)

    @pl.when(r == 0)
    def _init_out():
        out_ref[...] = jnp.zeros_like(out_ref)

    dims = (((1,), (1,)), ((), ()))
    topo = jax.lax.dot_general(
        tr_ref[...], tc_ref[...], dims,
        preferred_element_type=jnp.float32).astype(jnp.bfloat16)

    neg = jnp.asarray(_NEG, dtype=topo.dtype)
    prev = jnp.max(topo, axis=1, keepdims=True)
    for _ in range(_K):
        masked = jnp.where(topo < prev, topo, neg)
        prev = jnp.max(masked, axis=1, keepdims=True)
    t8 = prev

    zr = zr_ref[...]
    logits = jax.lax.dot_general(
        zr, zc_ref[...], dims, preferred_element_type=jnp.float32)

    lii = _TEMP_INV * jnp.sum(zr * zr, axis=1, keepdims=True)

    srow = jnp.sum(jnp.exp(logits), axis=1, keepdims=True) - jnp.exp(lii)
    prow = (jnp.sum(jnp.where(topo >= t8, logits, 0.0), axis=1, keepdims=True)
            - lii)
    lse = jnp.log(srow)
    part = jnp.sum(lse - prow / _K, axis=0, keepdims=True)
    out_ref[...] += part / batch


def kernel(projections, topo_vectors):
    B, D = projections.shape
    Dt = topo_vectors.shape[1]

    zrow, zcol, topo_z = pl.pallas_call(
        _normalize_body,
        out_shape=[
            jax.ShapeDtypeStruct((B, D), jnp.float32),
            jax.ShapeDtypeStruct((B, D), jnp.float32),
            jax.ShapeDtypeStruct((B, Dt), jnp.bfloat16),
        ],
    )(projections, topo_vectors)

    rtile = 512
    nr = B // rtile

    out = pl.pallas_call(
        functools.partial(_body, rtile=rtile, batch=float(B)),
        grid=(nr,),
        in_specs=[
            pl.BlockSpec((rtile, D), lambda r: (r, 0)),
            pl.BlockSpec((B, D), lambda r: (0, 0)),
            pl.BlockSpec((rtile, Dt), lambda r: (r, 0)),
            pl.BlockSpec((B, Dt), lambda r: (0, 0)),
        ],
        out_specs=pl.BlockSpec((1, 1), lambda r: (0, 0)),
        out_shape=jax.ShapeDtypeStruct((1, 1), jnp.float32),
        compiler_params=pltpu.CompilerParams(
            dimension_semantics=("arbitrary",)),
    )(zrow, zcol, topo_z, topo_z)
    return out[0, 0]


# speedup vs baseline: 30.3436x; 1.0608x over previous
import functools

import jax
import jax.numpy as jnp
from jax.experimental import pallas as pl
from jax.experimental.pallas import tpu as pltpu

_TEMP_INV = 10.0
_K = 8
_NEG = -3.0e38


def _normalize_body(z_ref, t_ref, zr_ref, zc_ref, to_ref):
    z = z_ref[...]
    t = t_ref[...]
    zn = jnp.maximum(jnp.sqrt(jnp.sum(z * z, axis=1, keepdims=True)), 1e-12)
    tn = jnp.maximum(jnp.sqrt(jnp.sum(t * t, axis=1, keepdims=True)), 1e-12)
    zr_ref[...] = z / zn
    zc_ref[...] = (z / zn) * _TEMP_INV
    to_ref[...] = (t / tn).astype(jnp.bfloat16)


def _body(zr_ref, zc_ref, tr_ref, tc_ref, out_ref, *, rtile, batch):
    r = pl.program_id(0)

    @pl.when(r == 0)
    def _init_out():
        out_ref[...] = jnp.zeros_like(out_ref)

    dims = (((1,), (1,)), ((), ()))
    topo = jax.lax.dot_general(
        tr_ref[...], tc_ref[...], dims,
        preferred_element_type=jnp.float32).astype(jnp.bfloat16)

    neg = jnp.asarray(_NEG, dtype=topo.dtype)
    prev = jnp.max(topo, axis=1, keepdims=True)
    for _ in range(_K):
        masked = jnp.where(topo < prev, topo, neg)
        prev = jnp.max(masked, axis=1, keepdims=True)
    t8 = prev

    zr = zr_ref[...]
    logits = jax.lax.dot_general(
        zr, zc_ref[...], dims, preferred_element_type=jnp.float32)

    lii = _TEMP_INV * jnp.sum(zr * zr, axis=1, keepdims=True)

    srow = jnp.sum(jnp.exp(logits), axis=1, keepdims=True) - jnp.exp(lii)
    prow = (jnp.sum(jnp.where(topo >= t8, logits, 0.0), axis=1, keepdims=True)
            - lii)
    lse = jnp.log(srow)
    part = jnp.sum(lse - prow / _K, axis=0, keepdims=True)
    out_ref[...] += part / batch


def kernel(projections, topo_vectors):
    B, D = projections.shape
    Dt = topo_vectors.shape[1]

    zrow, zcol, topo_z = pl.pallas_call(
        _normalize_body,
        out_shape=[
            jax.ShapeDtypeStruct((B, D), jnp.float32),
            jax.ShapeDtypeStruct((B, D), jnp.float32),
            jax.ShapeDtypeStruct((B, Dt), jnp.bfloat16),
        ],
    )(projections, topo_vectors)

    rtile = 1024
    nr = B // rtile

    out = pl.pallas_call(
        functools.partial(_body, rtile=rtile, batch=float(B)),
        grid=(nr,),
        in_specs=[
            pl.BlockSpec((rtile, D), lambda r: (r, 0)),
            pl.BlockSpec((B, D), lambda r: (0, 0)),
            pl.BlockSpec((rtile, Dt), lambda r: (r, 0)),
            pl.BlockSpec((B, Dt), lambda r: (0, 0)),
        ],
        out_specs=pl.BlockSpec((1, 1), lambda r: (0, 0)),
        out_shape=jax.ShapeDtypeStruct((1, 1), jnp.float32),
        compiler_params=pltpu.CompilerParams(
            dimension_semantics=("arbitrary",)),
    )(zrow, zcol, topo_z, topo_z)
    return out[0, 0]
